# baseline (speedup 1.0000x reference)
import functools

import jax
import jax.numpy as jnp
from jax import lax
from jax.experimental import pallas as pl
from jax.experimental.pallas import tpu as pltpu
from jax.experimental.pallas import tpu_sc as plsc

N = 10000
NPAD = 10240
E = 160000
EPAD = 163840
G = 64
NW = 32
ROWS_PER_W = EPAD // (NW * 128)
RPT = NPAD // 16

_mesh = plsc.VectorSubcoreMesh(core_axis_name="c", subcore_axis_name="s")


def _deg_body(dst_hbm, zeros_hbm, ones_hbm, out_hbm, idx_v, ones_v, acc, sem):
    cid = lax.axis_index("c")
    sid = lax.axis_index("s")
    wid = sid * 2 + cid
    pltpu.sync_copy(zeros_hbm.at[pl.ds(sid * RPT, RPT)],
                    acc.at[pl.ds(sid * RPT, RPT)])
    pltpu.sync_copy(ones_hbm, ones_v)
    pltpu.sync_copy(dst_hbm.at[pl.ds(wid * ROWS_PER_W, ROWS_PER_W)], idx_v)
    plsc.subcore_barrier()

    def body(j, carry):
        pltpu.sync_copy(ones_v, acc.at[idx_v.at[j]], add=True)
        return carry

    lax.fori_loop(0, ROWS_PER_W, body, 0)
    plsc.subcore_barrier()
    pltpu.sync_copy(acc.at[pl.ds(sid * RPT, RPT)],
                    out_hbm.at[cid, pl.ds(sid * RPT, RPT)])


def _sc_degree(dst2d, zeros, ones):
    return pl.kernel(
        _deg_body,
        out_type=jax.ShapeDtypeStruct((2, NPAD, 16), jnp.float32),
        mesh=_mesh,
        scratch_types=[
            pltpu.VMEM((ROWS_PER_W, 128), jnp.int32),
            pltpu.VMEM((128, 16), jnp.float32),
            pltpu.VMEM_SHARED((NPAD, 16), jnp.float32),
            pltpu.SemaphoreType.DMA,
        ],
        compiler_params=pltpu.CompilerParams(use_tc_tiling_on_sc=False),
    )(dst2d, zeros, ones)


_NBUF = 2


def _prop_body(hs_hbm, src_hbm, dst_hbm, out_hbm, isv, idv, rbufs, acc,
               sg, ss):
    cid = lax.axis_index("c")
    sid = lax.axis_index("s")
    wid = sid * 2 + cid
    pltpu.sync_copy(src_hbm.at[pl.ds(wid * ROWS_PER_W, ROWS_PER_W)], isv)
    pltpu.sync_copy(dst_hbm.at[pl.ds(wid * ROWS_PER_W, ROWS_PER_W)], idv)
    for b in range(_NBUF):
        pltpu.async_copy(hs_hbm.at[isv.at[b]], rbufs[b], sg[b])
    pltpu.sync_copy(hs_hbm.at[pl.ds(sid * RPT, RPT)],
                    acc.at[pl.ds(sid * RPT, RPT)])
    plsc.subcore_barrier()

    def body(i, carry):
        j = _NBUF * i
        for b in range(_NBUF):
            pltpu.make_async_copy(hs_hbm.at[isv.at[j + b]], rbufs[b],
                                  sg[b]).wait()
            pltpu.async_copy(rbufs[b], acc.at[idv.at[j + b]], ss[b],
                             add=True)
        for b in range(_NBUF):
            pltpu.make_async_copy(rbufs[b], acc.at[idv.at[j + b]],
                                  ss[b]).wait()

            @pl.when(j + b + _NBUF < ROWS_PER_W)
            def _():
                pltpu.async_copy(hs_hbm.at[isv.at[j + b + _NBUF]],
                                 rbufs[b], sg[b])
        return carry

    lax.fori_loop(0, ROWS_PER_W // _NBUF, body, 0)
    plsc.subcore_barrier()
    pltpu.sync_copy(acc.at[pl.ds(sid * RPT, RPT)],
                    out_hbm.at[cid, pl.ds(sid * RPT, RPT)])


def _sc_prop(hs, src2d, dst2d, D, tc_tiling):
    params = (None if tc_tiling
              else pltpu.CompilerParams(use_tc_tiling_on_sc=False))
    return pl.kernel(
        _prop_body,
        out_type=jax.ShapeDtypeStruct((2, NPAD, D), jnp.float32),
        mesh=_mesh,
        scratch_types=[
            pltpu.VMEM((ROWS_PER_W, 128), jnp.int32),
            pltpu.VMEM((ROWS_PER_W, 128), jnp.int32),
            [pltpu.VMEM((128, D), jnp.float32) for _ in range(_NBUF)],
            pltpu.VMEM_SHARED((NPAD, D), jnp.float32),
            [pltpu.SemaphoreType.DMA for _ in range(_NBUF)],
            [pltpu.SemaphoreType.DMA for _ in range(_NBUF)],
        ],
        compiler_params=params,
    )(hs, src2d, dst2d)


def _mm_body(x_ref, w_ref, out_ref):
    out_ref[...] = jnp.dot(x_ref[...], w_ref[...],
                           preferred_element_type=jnp.float32)


def _tc_matmul(xp, W1):
    return pl.pallas_call(
        _mm_body,
        out_shape=jax.ShapeDtypeStruct((NPAD, 128), jnp.float32),
    )(xp, W1)


def _scale_body(h_ref, deg_ref, hs_ref, dinv_ref):
    deg = deg_ref[0, :, 0:1] + deg_ref[1, :, 0:1] + 1.0
    dinv = lax.rsqrt(deg)
    hs_ref[...] = h_ref[...] * dinv
    dinv_ref[...] = dinv


def _tc_scale(h1, degp):
    return pl.pallas_call(
        _scale_body,
        out_shape=[
            jax.ShapeDtypeStruct((NPAD, 128), jnp.float32),
            jax.ShapeDtypeStruct((NPAD, 1), jnp.float32),
        ],
    )(h1, degp)


def _tc_mid_body(a_ref, hs_ref, dinv_ref, b_ref, w_ref, out_ref):
    dinv = dinv_ref[...]
    p = jnp.tanh(dinv * (a_ref[0] + a_ref[1] - hs_ref[...]) + b_ref[...])
    out_ref[...] = jnp.dot(p, w_ref[...],
                           preferred_element_type=jnp.float32) * dinv


def _tc_mid(acc, hs, dinv, b, W, Dout):
    return pl.pallas_call(
        _tc_mid_body,
        out_shape=jax.ShapeDtypeStruct((NPAD, Dout), jnp.float32),
    )(acc, hs, dinv, b, W)


def _tc_final_body(a_ref, hs_ref, dinv_ref, b_ref, batch_ref, mask_ref,
                   wc_ref, bc_ref, out_ref, h_ref):
    dinv = dinv_ref[...]
    p = jnp.tanh(dinv * (a_ref[0] + a_ref[1] - hs_ref[...]) + b_ref[...])
    paug = jnp.concatenate([p, jnp.ones((NPAD, 1), jnp.float32)], axis=1)
    iota = lax.broadcasted_iota(jnp.int32, (NPAD, G), 1)
    onehot = (batch_ref[...] == iota).astype(jnp.float32)
    s = lax.dot_general(onehot, paug, (((0,), (0,)), ((), ())),
                        preferred_element_type=jnp.float32)
    hp = s[:, 0:16] / jnp.maximum(s[:, 16:17], 1.0)
    hd = mask_ref[...] * (2.0 * hp)
    h_ref[...] = hd
    out_ref[...] = jnp.dot(hd, wc_ref[...],
                           preferred_element_type=jnp.float32) + bc_ref[...]


def _tc_final(acc, hs, dinv, b, batch2d, mask, Wc, bc):
    return pl.pallas_call(
        _tc_final_body,
        out_shape=[
            jax.ShapeDtypeStruct((G, 2), jnp.float32),
            jax.ShapeDtypeStruct((G, 16), jnp.float32),
        ],
    )(acc, hs, dinv, b, batch2d, mask, Wc, bc)


def kernel(x, edge_index, batch, W1, b1, W2, b2, W3, b3, Wc, bc):
    xp = jnp.zeros((NPAD, 256), jnp.float32).at[:N].set(x)
    src = edge_index[0].astype(jnp.int32)
    dst = edge_index[1].astype(jnp.int32)
    npe = EPAD - E
    pad_idx = N + (jnp.arange(npe, dtype=jnp.int32) % (NPAD - N))
    src2d = jnp.concatenate([src, pad_idx]).reshape(EPAD // 128, 128)
    dst2d = jnp.concatenate([dst, pad_idx]).reshape(EPAD // 128, 128)
    batch2d = jnp.concatenate(
        [batch.astype(jnp.int32),
         jnp.full((NPAD - N,), G, jnp.int32)]).reshape(NPAD, 1)
    zeros16 = jnp.zeros((NPAD, 16), jnp.float32)
    ones16 = jnp.ones((128, 16), jnp.float32)
    mask = jax.random.bernoulli(jax.random.key(42), 0.5,
                                (G, 16)).astype(jnp.float32)
    b1r = b1.reshape(1, 128)
    b2r = b2.reshape(1, 64)
    b3r = b3.reshape(1, 16)
    bcr = bc.reshape(1, 2)

    degp = _sc_degree(dst2d, zeros16, ones16)
    h1 = _tc_matmul(xp, W1)
    hs1, dinv = _tc_scale(h1, degp)
    acc1 = _sc_prop(hs1, src2d, dst2d, 128, tc_tiling=True)
    hs2 = _tc_mid(acc1, hs1, dinv, b1r, W2, 64)
    acc2 = _sc_prop(hs2, src2d, dst2d, 64, tc_tiling=False)
    hs3 = _tc_mid(acc2, hs2, dinv, b2r, W3, 16)
    acc3 = _sc_prop(hs3, src2d, dst2d, 16, tc_tiling=False)
    out2d, h = _tc_final(acc3, hs3, dinv, b3r, batch2d, mask, Wc, bcr)
    return (out2d.reshape(-1), h)


# speedup vs baseline: 1.1082x; 1.1082x over previous
import functools

import jax
import jax.numpy as jnp
from jax import lax
from jax.experimental import pallas as pl
from jax.experimental.pallas import tpu as pltpu
from jax.experimental.pallas import tpu_sc as plsc

N = 10000
NPAD = 10240
E = 160000
EPAD = 163840
G = 64
NW = 32
ROWS_PER_W = EPAD // (NW * 128)
RPT = NPAD // 16

_mesh = plsc.VectorSubcoreMesh(core_axis_name="c", subcore_axis_name="s")


def _deg_body(dst_hbm, zeros_hbm, ones_hbm, out_hbm, idx_v, ones_v, acc, sem):
    cid = lax.axis_index("c")
    sid = lax.axis_index("s")
    wid = sid * 2 + cid
    pltpu.sync_copy(zeros_hbm.at[pl.ds(sid * RPT, RPT)],
                    acc.at[pl.ds(sid * RPT, RPT)])
    pltpu.sync_copy(ones_hbm, ones_v)
    pltpu.sync_copy(dst_hbm.at[pl.ds(wid * ROWS_PER_W, ROWS_PER_W)], idx_v)
    plsc.subcore_barrier()

    def body(j, carry):
        pltpu.sync_copy(ones_v, acc.at[idx_v.at[j]], add=True)
        return carry

    lax.fori_loop(0, ROWS_PER_W, body, 0)
    plsc.subcore_barrier()
    pltpu.sync_copy(acc.at[pl.ds(sid * RPT, RPT)],
                    out_hbm.at[cid, pl.ds(sid * RPT, RPT)])


def _sc_degree(dst2d, zeros, ones):
    return pl.kernel(
        _deg_body,
        out_type=jax.ShapeDtypeStruct((2, NPAD, 16), jnp.float32),
        mesh=_mesh,
        scratch_types=[
            pltpu.VMEM((ROWS_PER_W, 128), jnp.int32),
            pltpu.VMEM((128, 16), jnp.float32),
            pltpu.VMEM_SHARED((NPAD, 16), jnp.float32),
            pltpu.SemaphoreType.DMA,
        ],
        compiler_params=pltpu.CompilerParams(use_tc_tiling_on_sc=False),
    )(dst2d, zeros, ones)


def _prop_body(hs_hbm, src_hbm, dst_hbm, out_hbm, isv, idv, r0, r1, acc,
               s0, s1):
    cid = lax.axis_index("c")
    sid = lax.axis_index("s")
    wid = sid * 2 + cid
    pltpu.sync_copy(src_hbm.at[pl.ds(wid * ROWS_PER_W, ROWS_PER_W)], isv)
    pltpu.sync_copy(dst_hbm.at[pl.ds(wid * ROWS_PER_W, ROWS_PER_W)], idv)
    pltpu.async_copy(hs_hbm.at[isv.at[0]], r0, s0)
    pltpu.sync_copy(hs_hbm.at[pl.ds(sid * RPT, RPT)],
                    acc.at[pl.ds(sid * RPT, RPT)])
    plsc.subcore_barrier()

    def body(i, carry):
        j0 = 2 * i
        j1 = 2 * i + 1
        cp1 = pltpu.async_copy(hs_hbm.at[isv.at[j1]], r1, s1)
        pltpu.make_async_copy(hs_hbm.at[isv.at[j0]], r0, s0).wait()
        pltpu.sync_copy(r0, acc.at[idv.at[j0]], add=True)

        @pl.when(j0 + 2 < ROWS_PER_W)
        def _():
            pltpu.async_copy(hs_hbm.at[isv.at[j0 + 2]], r0, s0)

        cp1.wait()
        pltpu.sync_copy(r1, acc.at[idv.at[j1]], add=True)
        return carry

    lax.fori_loop(0, ROWS_PER_W // 2, body, 0)
    plsc.subcore_barrier()
    pltpu.sync_copy(acc.at[pl.ds(sid * RPT, RPT)],
                    out_hbm.at[cid, pl.ds(sid * RPT, RPT)])


def _sc_prop(hs, src2d, dst2d, D, tc_tiling):
    params = (None if tc_tiling
              else pltpu.CompilerParams(use_tc_tiling_on_sc=False))
    return pl.kernel(
        _prop_body,
        out_type=jax.ShapeDtypeStruct((2, NPAD, D), jnp.float32),
        mesh=_mesh,
        scratch_types=[
            pltpu.VMEM((ROWS_PER_W, 128), jnp.int32),
            pltpu.VMEM((ROWS_PER_W, 128), jnp.int32),
            pltpu.VMEM((128, D), jnp.float32),
            pltpu.VMEM((128, D), jnp.float32),
            pltpu.VMEM_SHARED((NPAD, D), jnp.float32),
            pltpu.SemaphoreType.DMA,
            pltpu.SemaphoreType.DMA,
        ],
        compiler_params=params,
    )(hs, src2d, dst2d)


def _mm_body(x_ref, w_ref, out_ref):
    out_ref[...] = jnp.dot(x_ref[...], w_ref[...],
                           preferred_element_type=jnp.float32)


def _tc_matmul(xp, W1):
    return pl.pallas_call(
        _mm_body,
        out_shape=jax.ShapeDtypeStruct((NPAD, 128), jnp.float32),
    )(xp, W1)


def _scale_body(h_ref, deg_ref, hs_ref, dinv_ref):
    deg = deg_ref[0, :, 0:1] + deg_ref[1, :, 0:1] + 1.0
    dinv = lax.rsqrt(deg)
    hs_ref[...] = h_ref[...] * dinv
    dinv_ref[...] = dinv


def _tc_scale(h1, degp):
    return pl.pallas_call(
        _scale_body,
        out_shape=[
            jax.ShapeDtypeStruct((NPAD, 128), jnp.float32),
            jax.ShapeDtypeStruct((NPAD, 1), jnp.float32),
        ],
    )(h1, degp)


def _tc_mid_body(a_ref, hs_ref, dinv_ref, b_ref, w_ref, out_ref):
    dinv = dinv_ref[...]
    p = jnp.tanh(dinv * (a_ref[0] + a_ref[1] - hs_ref[...]) + b_ref[...])
    out_ref[...] = jnp.dot(p, w_ref[...],
                           preferred_element_type=jnp.float32) * dinv


def _tc_mid(acc, hs, dinv, b, W, Dout):
    return pl.pallas_call(
        _tc_mid_body,
        out_shape=jax.ShapeDtypeStruct((NPAD, Dout), jnp.float32),
    )(acc, hs, dinv, b, W)


def _tc_final_body(a_ref, hs_ref, dinv_ref, b_ref, batch_ref, mask_ref,
                   wc_ref, bc_ref, out_ref, h_ref):
    dinv = dinv_ref[...]
    p = jnp.tanh(dinv * (a_ref[0] + a_ref[1] - hs_ref[...]) + b_ref[...])
    paug = jnp.concatenate([p, jnp.ones((NPAD, 1), jnp.float32)], axis=1)
    iota = lax.broadcasted_iota(jnp.int32, (NPAD, G), 1)
    onehot = (batch_ref[...] == iota).astype(jnp.float32)
    s = lax.dot_general(onehot, paug, (((0,), (0,)), ((), ())),
                        preferred_element_type=jnp.float32)
    hp = s[:, 0:16] / jnp.maximum(s[:, 16:17], 1.0)
    hd = mask_ref[...] * (2.0 * hp)
    h_ref[...] = hd
    out_ref[...] = jnp.dot(hd, wc_ref[...],
                           preferred_element_type=jnp.float32) + bc_ref[...]


def _tc_final(acc, hs, dinv, b, batch2d, mask, Wc, bc):
    return pl.pallas_call(
        _tc_final_body,
        out_shape=[
            jax.ShapeDtypeStruct((G, 2), jnp.float32),
            jax.ShapeDtypeStruct((G, 16), jnp.float32),
        ],
    )(acc, hs, dinv, b, batch2d, mask, Wc, bc)


def kernel(x, edge_index, batch, W1, b1, W2, b2, W3, b3, Wc, bc):
    xp = jnp.zeros((NPAD, 256), jnp.float32).at[:N].set(x)
    src = edge_index[0].astype(jnp.int32)
    dst = edge_index[1].astype(jnp.int32)
    npe = EPAD - E
    pad_idx = N + (jnp.arange(npe, dtype=jnp.int32) % (NPAD - N))
    src2d = jnp.concatenate([src, pad_idx]).reshape(EPAD // 128, 128)
    dst2d = jnp.concatenate([dst, pad_idx]).reshape(EPAD // 128, 128)
    batch2d = jnp.concatenate(
        [batch.astype(jnp.int32),
         jnp.full((NPAD - N,), G, jnp.int32)]).reshape(NPAD, 1)
    zeros16 = jnp.zeros((NPAD, 16), jnp.float32)
    ones16 = jnp.ones((128, 16), jnp.float32)
    mask = jax.random.bernoulli(jax.random.key(42), 0.5,
                                (G, 16)).astype(jnp.float32)
    b1r = b1.reshape(1, 128)
    b2r = b2.reshape(1, 64)
    b3r = b3.reshape(1, 16)
    bcr = bc.reshape(1, 2)

    degp = _sc_degree(dst2d, zeros16, ones16)
    h1 = _tc_matmul(xp, W1)
    hs1, dinv = _tc_scale(h1, degp)
    acc1 = _sc_prop(hs1, src2d, dst2d, 128, tc_tiling=True)
    hs2 = _tc_mid(acc1, hs1, dinv, b1r, W2, 64)
    acc2 = _sc_prop(hs2, src2d, dst2d, 64, tc_tiling=False)
    hs3 = _tc_mid(acc2, hs2, dinv, b2r, W3, 16)
    acc3 = _sc_prop(hs3, src2d, dst2d, 16, tc_tiling=False)
    out2d, h = _tc_final(acc3, hs3, dinv, b3r, batch2d, mask, Wc, bcr)
    return (out2d.reshape(-1), h)


# speedup vs baseline: 1.1439x; 1.0322x over previous
import functools

import jax
import jax.numpy as jnp
from jax import lax
from jax.experimental import pallas as pl
from jax.experimental.pallas import tpu as pltpu
from jax.experimental.pallas import tpu_sc as plsc

N = 10000
NPAD = 10240
E = 160000
EROWS = E // 128
G = 64
NW = 32
ROWS_PER_W = 40
RPT = NPAD // 16


def _worker_rows(wid):
    nrows = jnp.where(wid < 2, 40, 39)
    base = jnp.where(wid < 2, wid * 40, 80 + (wid - 2) * 39)
    return base, nrows

_mesh = plsc.VectorSubcoreMesh(core_axis_name="c", subcore_axis_name="s")


def _load_worker_rows(ei3_hbm, eiv, base):
    @pl.when(base < 80)
    def _():
        pltpu.sync_copy(ei3_hbm.at[pl.ds(base, 40)], eiv)

    @pl.when(base >= 80)
    def _():
        pltpu.sync_copy(ei3_hbm.at[pl.ds(base, 39)], eiv.at[pl.ds(0, 39)])


def _deg_body(ei3_hbm, zeros_hbm, ones_hbm, out_hbm, eiv, ones_v, acc, sem):
    cid = lax.axis_index("c")
    sid = lax.axis_index("s")
    wid = sid * 2 + cid
    base, nrows = _worker_rows(wid)
    pltpu.sync_copy(zeros_hbm.at[pl.ds(sid * RPT, RPT)],
                    acc.at[pl.ds(sid * RPT, RPT)])
    pltpu.sync_copy(ones_hbm, ones_v)
    _load_worker_rows(ei3_hbm, eiv, base)
    plsc.subcore_barrier()

    def body(j, carry):
        pltpu.sync_copy(ones_v, acc.at[eiv.at[j, 1]], add=True)
        return carry

    lax.fori_loop(0, nrows, body, 0)
    plsc.subcore_barrier()
    pltpu.sync_copy(acc.at[pl.ds(sid * RPT, RPT)],
                    out_hbm.at[cid, pl.ds(sid * RPT, RPT)])


def _sc_degree(ei3, zeros, ones):
    return pl.kernel(
        _deg_body,
        out_type=jax.ShapeDtypeStruct((2, NPAD, 16), jnp.float32),
        mesh=_mesh,
        scratch_types=[
            pltpu.VMEM((ROWS_PER_W, 2, 128), jnp.int32),
            pltpu.VMEM((128, 16), jnp.float32),
            pltpu.VMEM_SHARED((NPAD, 16), jnp.float32),
            pltpu.SemaphoreType.DMA,
        ],
        compiler_params=pltpu.CompilerParams(use_tc_tiling_on_sc=False),
    )(ei3, zeros, ones)


def _prop_body(hs_hbm, ei3_hbm, out_hbm, eiv, r0, r1, acc, s0, s1):
    cid = lax.axis_index("c")
    sid = lax.axis_index("s")
    wid = sid * 2 + cid
    base, nrows = _worker_rows(wid)
    _load_worker_rows(ei3_hbm, eiv, base)
    pltpu.async_copy(hs_hbm.at[eiv.at[0, 0]], r0, s0)
    pltpu.sync_copy(hs_hbm.at[pl.ds(sid * RPT, RPT)],
                    acc.at[pl.ds(sid * RPT, RPT)])
    plsc.subcore_barrier()

    def body(i, carry):
        j0 = 2 * i
        j1 = 2 * i + 1

        @pl.when(j1 < nrows)
        def _():
            pltpu.async_copy(hs_hbm.at[eiv.at[j1, 0]], r1, s1)

        pltpu.make_async_copy(hs_hbm.at[eiv.at[j0, 0]], r0, s0).wait()
        pltpu.sync_copy(r0, acc.at[eiv.at[j0, 1]], add=True)

        @pl.when(j0 + 2 < nrows)
        def _():
            pltpu.async_copy(hs_hbm.at[eiv.at[j0 + 2, 0]], r0, s0)

        @pl.when(j1 < nrows)
        def _():
            pltpu.make_async_copy(hs_hbm.at[eiv.at[j1, 0]], r1, s1).wait()
            pltpu.sync_copy(r1, acc.at[eiv.at[j1, 1]], add=True)

        return carry

    lax.fori_loop(0, ROWS_PER_W // 2, body, 0)
    plsc.subcore_barrier()
    pltpu.sync_copy(acc.at[pl.ds(sid * RPT, RPT)],
                    out_hbm.at[cid, pl.ds(sid * RPT, RPT)])


def _sc_prop(hs, ei3, D):
    return pl.kernel(
        _prop_body,
        out_type=jax.ShapeDtypeStruct((2, NPAD, D), jnp.float32),
        mesh=_mesh,
        scratch_types=[
            pltpu.VMEM((ROWS_PER_W, 2, 128), jnp.int32),
            pltpu.VMEM((128, D), jnp.float32),
            pltpu.VMEM((128, D), jnp.float32),
            pltpu.VMEM_SHARED((NPAD, D), jnp.float32),
            pltpu.SemaphoreType.DMA,
            pltpu.SemaphoreType.DMA,
        ],
        compiler_params=pltpu.CompilerParams(use_tc_tiling_on_sc=False),
    )(hs, ei3)


def _mm_body(x_ref, w_ref, out_ref):
    out_ref[...] = jnp.dot(x_ref[...], w_ref[...],
                           preferred_element_type=jnp.float32)


def _tc_matmul(xp, W1):
    return pl.pallas_call(
        _mm_body,
        out_shape=jax.ShapeDtypeStruct((NPAD, 128), jnp.float32),
    )(xp, W1)


def _scale_body(h_ref, deg_ref, hs_ref, dinv_ref):
    deg = deg_ref[0, :, 0:1] + deg_ref[1, :, 0:1] + 1.0
    dinv = lax.rsqrt(deg)
    hs_ref[...] = h_ref[...] * dinv
    dinv_ref[...] = dinv


def _tc_scale(h1, degp):
    return pl.pallas_call(
        _scale_body,
        out_shape=[
            jax.ShapeDtypeStruct((NPAD, 128), jnp.float32),
            jax.ShapeDtypeStruct((NPAD, 1), jnp.float32),
        ],
    )(h1, degp)


def _tc_mid_body(a_ref, hs_ref, dinv_ref, b_ref, w_ref, out_ref):
    dinv = dinv_ref[...]
    p = jnp.tanh(dinv * (a_ref[0] + a_ref[1] - hs_ref[...]) + b_ref[...])
    out_ref[...] = jnp.dot(p, w_ref[...],
                           preferred_element_type=jnp.float32) * dinv


def _tc_mid(acc, hs, dinv, b, W, Dout):
    return pl.pallas_call(
        _tc_mid_body,
        out_shape=jax.ShapeDtypeStruct((NPAD, Dout), jnp.float32),
    )(acc, hs, dinv, b, W)


def _tc_final_body(a_ref, hs_ref, dinv_ref, b_ref, batch_ref, mask_ref,
                   wc_ref, bc_ref, out_ref, h_ref):
    dinv = dinv_ref[...]
    p = jnp.tanh(dinv * (a_ref[0] + a_ref[1] - hs_ref[...]) + b_ref[...])
    paug = jnp.concatenate([p, jnp.ones((NPAD, 1), jnp.float32)], axis=1)
    iota = lax.broadcasted_iota(jnp.int32, (NPAD, G), 1)
    onehot = (batch_ref[...] == iota).astype(jnp.float32)
    s = lax.dot_general(onehot, paug, (((0,), (0,)), ((), ())),
                        preferred_element_type=jnp.float32)
    hp = s[:, 0:16] / jnp.maximum(s[:, 16:17], 1.0)
    hd = mask_ref[...] * (2.0 * hp)
    h_ref[...] = hd
    out_ref[...] = jnp.dot(hd, wc_ref[...],
                           preferred_element_type=jnp.float32) + bc_ref[...]


def _tc_final(acc, hs, dinv, b, batch2d, mask, Wc, bc):
    return pl.pallas_call(
        _tc_final_body,
        out_shape=[
            jax.ShapeDtypeStruct((G, 2), jnp.float32),
            jax.ShapeDtypeStruct((G, 16), jnp.float32),
        ],
    )(acc, hs, dinv, b, batch2d, mask, Wc, bc)


def kernel(x, edge_index, batch, W1, b1, W2, b2, W3, b3, Wc, bc):
    xp = jnp.zeros((NPAD, 256), jnp.float32).at[:N].set(x)
    ei3 = jnp.transpose(
        edge_index.astype(jnp.int32).reshape(2, EROWS, 128), (1, 0, 2))
    batch2d = jnp.concatenate(
        [batch.astype(jnp.int32),
         jnp.full((NPAD - N,), G, jnp.int32)]).reshape(NPAD, 1)
    zeros16 = jnp.zeros((NPAD, 16), jnp.float32)
    ones16 = jnp.ones((128, 16), jnp.float32)
    mask = jax.random.bernoulli(jax.random.key(42), 0.5,
                                (G, 16)).astype(jnp.float32)
    b1r = b1.reshape(1, 128)
    b2r = b2.reshape(1, 64)
    b3r = b3.reshape(1, 16)
    bcr = bc.reshape(1, 2)

    degp = _sc_degree(ei3, zeros16, ones16)
    h1 = _tc_matmul(xp, W1)
    hs1, dinv = _tc_scale(h1, degp)
    acc1 = _sc_prop(hs1, ei3, 128)
    hs2 = _tc_mid(acc1, hs1, dinv, b1r, W2, 64)
    acc2 = _sc_prop(hs2, ei3, 64)
    hs3 = _tc_mid(acc2, hs2, dinv, b2r, W3, 16)
    acc3 = _sc_prop(hs3, ei3, 16)
    out2d, h = _tc_final(acc3, hs3, dinv, b3r, batch2d, mask, Wc, bcr)
    return (out2d.reshape(-1), h)


# speedup vs baseline: 1.1532x; 1.0082x over previous
import functools

import jax
import jax.numpy as jnp
from jax import lax
from jax.experimental import pallas as pl
from jax.experimental.pallas import tpu as pltpu
from jax.experimental.pallas import tpu_sc as plsc

N = 10000
NPAD = 10240
E = 160000
EROWS = E // 128
G = 64
NW = 32
ROWS_PER_W = 40
RPT = NPAD // 16


def _worker_rows(wid):
    nrows = jnp.where(wid < 2, 40, 39)
    base = jnp.where(wid < 2, wid * 40, 80 + (wid - 2) * 39)
    return base, nrows

_mesh = plsc.VectorSubcoreMesh(core_axis_name="c", subcore_axis_name="s")


def _load_worker_rows(ei3_hbm, eiv, base):
    @pl.when(base < 80)
    def _():
        pltpu.sync_copy(ei3_hbm.at[pl.ds(base, 40)], eiv)

    @pl.when(base >= 80)
    def _():
        pltpu.sync_copy(ei3_hbm.at[pl.ds(base, 39)], eiv.at[pl.ds(0, 39)])


def _deg_body(ei3_hbm, zeros_hbm, ones_hbm, out_hbm, eiv, ones_v, acc, sem):
    cid = lax.axis_index("c")
    sid = lax.axis_index("s")
    wid = sid * 2 + cid
    base, nrows = _worker_rows(wid)
    pltpu.sync_copy(zeros_hbm.at[pl.ds(sid * RPT, RPT)],
                    acc.at[pl.ds(sid * RPT, RPT)])
    pltpu.sync_copy(ones_hbm, ones_v)
    _load_worker_rows(ei3_hbm, eiv, base)
    plsc.subcore_barrier()

    def body(j, carry):
        pltpu.sync_copy(ones_v, acc.at[eiv.at[j, 1]], add=True)
        return carry

    lax.fori_loop(0, nrows, body, 0)
    plsc.subcore_barrier()
    pltpu.sync_copy(acc.at[pl.ds(sid * RPT, RPT)],
                    out_hbm.at[cid, pl.ds(sid * RPT, RPT)])


def _sc_degree(ei3, zeros, ones):
    return pl.kernel(
        _deg_body,
        out_type=jax.ShapeDtypeStruct((2, NPAD, 16), jnp.float32),
        mesh=_mesh,
        scratch_types=[
            pltpu.VMEM((ROWS_PER_W, 2, 128), jnp.int32),
            pltpu.VMEM((128, 16), jnp.float32),
            pltpu.VMEM_SHARED((NPAD, 16), jnp.float32),
            pltpu.SemaphoreType.DMA,
        ],
        compiler_params=pltpu.CompilerParams(use_tc_tiling_on_sc=False),
    )(ei3, zeros, ones)


def _prop_body(hs_hbm, ei3_hbm, out_hbm, eiv, r0, r1, acc, s0, s1):
    cid = lax.axis_index("c")
    sid = lax.axis_index("s")
    wid = sid * 2 + cid
    base, nrows = _worker_rows(wid)
    _load_worker_rows(ei3_hbm, eiv, base)
    pltpu.async_copy(hs_hbm.at[eiv.at[0, 0]], r0, s0)
    pltpu.sync_copy(hs_hbm.at[pl.ds(sid * RPT, RPT)],
                    acc.at[pl.ds(sid * RPT, RPT)])
    plsc.subcore_barrier()

    def body(i, carry):
        j0 = 2 * i
        j1 = 2 * i + 1

        @pl.when(j1 < nrows)
        def _():
            pltpu.async_copy(hs_hbm.at[eiv.at[j1, 0]], r1, s1)

        pltpu.make_async_copy(hs_hbm.at[eiv.at[j0, 0]], r0, s0).wait()
        pltpu.sync_copy(r0, acc.at[eiv.at[j0, 1]], add=True)

        @pl.when(j0 + 2 < nrows)
        def _():
            pltpu.async_copy(hs_hbm.at[eiv.at[j0 + 2, 0]], r0, s0)

        @pl.when(j1 < nrows)
        def _():
            pltpu.make_async_copy(hs_hbm.at[eiv.at[j1, 0]], r1, s1).wait()
            pltpu.sync_copy(r1, acc.at[eiv.at[j1, 1]], add=True)

        return carry

    lax.fori_loop(0, ROWS_PER_W // 2, body, 0)
    plsc.subcore_barrier()
    pltpu.sync_copy(acc.at[pl.ds(sid * RPT, RPT)],
                    out_hbm.at[cid, pl.ds(sid * RPT, RPT)])


def _sc_prop(hs, ei3, D):
    dt = hs.dtype
    return pl.kernel(
        _prop_body,
        out_type=jax.ShapeDtypeStruct((2, NPAD, D), dt),
        mesh=_mesh,
        scratch_types=[
            pltpu.VMEM((ROWS_PER_W, 2, 128), jnp.int32),
            pltpu.VMEM((128, D), dt),
            pltpu.VMEM((128, D), dt),
            pltpu.VMEM_SHARED((NPAD, D), dt),
            pltpu.SemaphoreType.DMA,
            pltpu.SemaphoreType.DMA,
        ],
        compiler_params=pltpu.CompilerParams(use_tc_tiling_on_sc=False),
    )(hs, ei3)


def _mm_body(x_ref, w_ref, out_ref):
    out_ref[...] = jnp.dot(x_ref[...], w_ref[...],
                           preferred_element_type=jnp.float32)


def _tc_matmul(xp, W1):
    return pl.pallas_call(
        _mm_body,
        out_shape=jax.ShapeDtypeStruct((NPAD, 128), jnp.float32),
    )(xp, W1)


def _scale_body(h_ref, deg_ref, hs_ref, dinv_ref):
    deg = deg_ref[0, :, 0:1] + deg_ref[1, :, 0:1] + 1.0
    dinv = lax.rsqrt(deg)
    hs_ref[...] = (h_ref[...] * dinv).astype(jnp.bfloat16)
    dinv_ref[...] = dinv


def _tc_scale(h1, degp):
    return pl.pallas_call(
        _scale_body,
        out_shape=[
            jax.ShapeDtypeStruct((NPAD, 128), jnp.bfloat16),
            jax.ShapeDtypeStruct((NPAD, 1), jnp.float32),
        ],
    )(h1, degp)


def _tc_mid_body(a_ref, hs_ref, dinv_ref, b_ref, w_ref, out_ref):
    dinv = dinv_ref[...]
    a = a_ref[0].astype(jnp.float32) + a_ref[1].astype(jnp.float32)
    p = jnp.tanh(dinv * (a - hs_ref[...].astype(jnp.float32)) + b_ref[...])
    out = jnp.dot(p, w_ref[...], preferred_element_type=jnp.float32) * dinv
    out_ref[...] = out.astype(out_ref.dtype)


def _tc_mid(acc, hs, dinv, b, W, Dout, out_dtype):
    return pl.pallas_call(
        _tc_mid_body,
        out_shape=jax.ShapeDtypeStruct((NPAD, Dout), out_dtype),
    )(acc, hs, dinv, b, W)


def _tc_final_body(a_ref, hs_ref, dinv_ref, b_ref, batch_ref, mask_ref,
                   wc_ref, bc_ref, out_ref, h_ref):
    dinv = dinv_ref[...]
    p = jnp.tanh(dinv * (a_ref[0] + a_ref[1] - hs_ref[...]) + b_ref[...])
    paug = jnp.concatenate([p, jnp.ones((NPAD, 1), jnp.float32)], axis=1)
    iota = lax.broadcasted_iota(jnp.int32, (NPAD, G), 1)
    onehot = (batch_ref[...] == iota).astype(jnp.float32)
    s = lax.dot_general(onehot, paug, (((0,), (0,)), ((), ())),
                        preferred_element_type=jnp.float32)
    hp = s[:, 0:16] / jnp.maximum(s[:, 16:17], 1.0)
    hd = mask_ref[...] * (2.0 * hp)
    h_ref[...] = hd
    out_ref[...] = jnp.dot(hd, wc_ref[...],
                           preferred_element_type=jnp.float32) + bc_ref[...]


def _tc_final(acc, hs, dinv, b, batch2d, mask, Wc, bc):
    return pl.pallas_call(
        _tc_final_body,
        out_shape=[
            jax.ShapeDtypeStruct((G, 2), jnp.float32),
            jax.ShapeDtypeStruct((G, 16), jnp.float32),
        ],
    )(acc, hs, dinv, b, batch2d, mask, Wc, bc)


def kernel(x, edge_index, batch, W1, b1, W2, b2, W3, b3, Wc, bc):
    xp = jnp.zeros((NPAD, 256), jnp.float32).at[:N].set(x)
    ei3 = jnp.transpose(
        edge_index.astype(jnp.int32).reshape(2, EROWS, 128), (1, 0, 2))
    batch2d = jnp.concatenate(
        [batch.astype(jnp.int32),
         jnp.full((NPAD - N,), G, jnp.int32)]).reshape(NPAD, 1)
    zeros16 = jnp.zeros((NPAD, 16), jnp.float32)
    ones16 = jnp.ones((128, 16), jnp.float32)
    mask = jax.random.bernoulli(jax.random.key(42), 0.5,
                                (G, 16)).astype(jnp.float32)
    b1r = b1.reshape(1, 128)
    b2r = b2.reshape(1, 64)
    b3r = b3.reshape(1, 16)
    bcr = bc.reshape(1, 2)

    degp = _sc_degree(ei3, zeros16, ones16)
    h1 = _tc_matmul(xp, W1)
    hs1, dinv = _tc_scale(h1, degp)
    acc1 = _sc_prop(hs1, ei3, 128)
    hs2 = _tc_mid(acc1, hs1, dinv, b1r, W2, 64, jnp.bfloat16)
    acc2 = _sc_prop(hs2, ei3, 64)
    hs3 = _tc_mid(acc2, hs2, dinv, b2r, W3, 16, jnp.float32)
    acc3 = _sc_prop(hs3, ei3, 16)
    out2d, h = _tc_final(acc3, hs3, dinv, b3r, batch2d, mask, Wc, bcr)
    return (out2d.reshape(-1), h)


# speedup vs baseline: 1.2868x; 1.1159x over previous
import functools

import jax
import jax.numpy as jnp
from jax import lax
from jax.experimental import pallas as pl
from jax.experimental.pallas import tpu as pltpu
from jax.experimental.pallas import tpu_sc as plsc

N = 10000
NPAD = 10240
E = 160000
EROWS = E // 128
G = 64
NW = 32
ROWS_PER_W = 40
RPT = NPAD // 16


def _worker_rows(wid):
    nrows = jnp.where(wid < 2, 40, 39)
    base = jnp.where(wid < 2, wid * 40, 80 + (wid - 2) * 39)
    return base, nrows

_mesh = plsc.VectorSubcoreMesh(core_axis_name="c", subcore_axis_name="s")


def _load_worker_rows(ei3_hbm, eiv, base):
    @pl.when(base < 80)
    def _():
        pltpu.sync_copy(ei3_hbm.at[pl.ds(base, 40)], eiv)

    @pl.when(base >= 80)
    def _():
        pltpu.sync_copy(ei3_hbm.at[pl.ds(base, 39)], eiv.at[pl.ds(0, 39)])


def _deg_body(ei3_hbm, zeros_hbm, ones_hbm, out_hbm, eiv, ones_v, acc, sem):
    cid = lax.axis_index("c")
    sid = lax.axis_index("s")
    wid = sid * 2 + cid
    base, nrows = _worker_rows(wid)
    pltpu.sync_copy(zeros_hbm.at[pl.ds(sid * RPT, RPT)],
                    acc.at[pl.ds(sid * RPT, RPT)])
    pltpu.sync_copy(ones_hbm, ones_v)
    _load_worker_rows(ei3_hbm, eiv, base)
    plsc.subcore_barrier()

    def body(j, carry):
        pltpu.sync_copy(ones_v, acc.at[eiv.at[j, 1]], add=True)
        return carry

    lax.fori_loop(0, nrows, body, 0)
    plsc.subcore_barrier()
    pltpu.sync_copy(acc.at[pl.ds(sid * RPT, RPT)],
                    out_hbm.at[cid, pl.ds(sid * RPT, RPT)])


def _sc_degree(ei3, zeros, ones):
    return pl.kernel(
        _deg_body,
        out_type=jax.ShapeDtypeStruct((2, NPAD, 16), jnp.float32),
        mesh=_mesh,
        scratch_types=[
            pltpu.VMEM((ROWS_PER_W, 2, 128), jnp.int32),
            pltpu.VMEM((128, 16), jnp.float32),
            pltpu.VMEM_SHARED((NPAD, 16), jnp.float32),
            pltpu.SemaphoreType.DMA,
        ],
        compiler_params=pltpu.CompilerParams(use_tc_tiling_on_sc=False),
    )(ei3, zeros, ones)


_NG = 4


def _prop_body(hs_hbm, ei3_hbm, out_hbm, eiv, rbufs, acc, sg):
    cid = lax.axis_index("c")
    sid = lax.axis_index("s")
    wid = sid * 2 + cid
    base, nrows = _worker_rows(wid)
    _load_worker_rows(ei3_hbm, eiv, base)
    for b in range(_NG):
        pltpu.async_copy(hs_hbm.at[eiv.at[b, 0]], rbufs[b], sg[b])
    pltpu.sync_copy(hs_hbm.at[pl.ds(sid * RPT, RPT)],
                    acc.at[pl.ds(sid * RPT, RPT)])
    plsc.subcore_barrier()

    def one_op(j, b):
        pltpu.make_async_copy(hs_hbm.at[eiv.at[j, 0]], rbufs[b],
                              sg[b]).wait()
        pltpu.sync_copy(rbufs[b], acc.at[eiv.at[j, 1]], add=True)

        @pl.when(j + _NG < nrows)
        def _():
            pltpu.async_copy(hs_hbm.at[eiv.at[j + _NG, 0]], rbufs[b], sg[b])

    def body(i, carry):
        for b in range(_NG):
            one_op(_NG * i + b, b)
        return carry

    lax.fori_loop(0, ROWS_PER_W // _NG - 1, body, 0)
    for b in range(_NG - 1):
        one_op(ROWS_PER_W - _NG + b, b)

    @pl.when(nrows >= ROWS_PER_W)
    def _():
        one_op(ROWS_PER_W - 1, _NG - 1)

    plsc.subcore_barrier()
    pltpu.sync_copy(acc.at[pl.ds(sid * RPT, RPT)],
                    out_hbm.at[cid, pl.ds(sid * RPT, RPT)])


def _sc_prop(hs, ei3, D):
    dt = hs.dtype
    return pl.kernel(
        _prop_body,
        out_type=jax.ShapeDtypeStruct((2, NPAD, D), dt),
        mesh=_mesh,
        scratch_types=[
            pltpu.VMEM((ROWS_PER_W, 2, 128), jnp.int32),
            [pltpu.VMEM((128, D), dt) for _ in range(_NG)],
            pltpu.VMEM_SHARED((NPAD, D), dt),
            [pltpu.SemaphoreType.DMA for _ in range(_NG)],
        ],
        compiler_params=pltpu.CompilerParams(use_tc_tiling_on_sc=False),
    )(hs, ei3)


def _mm_body(x_ref, w_ref, out_ref):
    out_ref[...] = jnp.dot(x_ref[...], w_ref[...],
                           preferred_element_type=jnp.float32)


def _tc_matmul(xp, W1):
    return pl.pallas_call(
        _mm_body,
        out_shape=jax.ShapeDtypeStruct((NPAD, 128), jnp.float32),
    )(xp, W1)


def _scale_body(h_ref, deg_ref, hs_ref, dinv_ref):
    deg = deg_ref[0, :, 0:1] + deg_ref[1, :, 0:1] + 1.0
    dinv = lax.rsqrt(deg)
    hs_ref[...] = (h_ref[...] * dinv).astype(jnp.bfloat16)
    dinv_ref[...] = dinv


def _tc_scale(h1, degp):
    return pl.pallas_call(
        _scale_body,
        out_shape=[
            jax.ShapeDtypeStruct((NPAD, 128), jnp.bfloat16),
            jax.ShapeDtypeStruct((NPAD, 1), jnp.float32),
        ],
    )(h1, degp)


def _tc_mid_body(a_ref, hs_ref, dinv_ref, b_ref, w_ref, out_ref):
    dinv = dinv_ref[...]
    a = a_ref[0].astype(jnp.float32) + a_ref[1].astype(jnp.float32)
    p = jnp.tanh(dinv * (a - hs_ref[...].astype(jnp.float32)) + b_ref[...])
    out = jnp.dot(p, w_ref[...], preferred_element_type=jnp.float32) * dinv
    out_ref[...] = out.astype(out_ref.dtype)


def _tc_mid(acc, hs, dinv, b, W, Dout, out_dtype):
    return pl.pallas_call(
        _tc_mid_body,
        out_shape=jax.ShapeDtypeStruct((NPAD, Dout), out_dtype),
    )(acc, hs, dinv, b, W)


def _tc_final_body(a_ref, hs_ref, dinv_ref, b_ref, batch_ref, mask_ref,
                   wc_ref, bc_ref, out_ref, h_ref):
    dinv = dinv_ref[...]
    p = jnp.tanh(dinv * (a_ref[0] + a_ref[1] - hs_ref[...]) + b_ref[...])
    paug = jnp.concatenate([p, jnp.ones((NPAD, 1), jnp.float32)], axis=1)
    iota = lax.broadcasted_iota(jnp.int32, (NPAD, G), 1)
    onehot = (batch_ref[...] == iota).astype(jnp.float32)
    s = lax.dot_general(onehot, paug, (((0,), (0,)), ((), ())),
                        preferred_element_type=jnp.float32)
    hp = s[:, 0:16] / jnp.maximum(s[:, 16:17], 1.0)
    hd = mask_ref[...] * (2.0 * hp)
    h_ref[...] = hd
    out_ref[...] = jnp.dot(hd, wc_ref[...],
                           preferred_element_type=jnp.float32) + bc_ref[...]


def _tc_final(acc, hs, dinv, b, batch2d, mask, Wc, bc):
    return pl.pallas_call(
        _tc_final_body,
        out_shape=[
            jax.ShapeDtypeStruct((G, 2), jnp.float32),
            jax.ShapeDtypeStruct((G, 16), jnp.float32),
        ],
    )(acc, hs, dinv, b, batch2d, mask, Wc, bc)


def kernel(x, edge_index, batch, W1, b1, W2, b2, W3, b3, Wc, bc):
    xp = jnp.zeros((NPAD, 256), jnp.float32).at[:N].set(x)
    ei3 = jnp.transpose(
        edge_index.astype(jnp.int32).reshape(2, EROWS, 128), (1, 0, 2))
    batch2d = jnp.concatenate(
        [batch.astype(jnp.int32),
         jnp.full((NPAD - N,), G, jnp.int32)]).reshape(NPAD, 1)
    zeros16 = jnp.zeros((NPAD, 16), jnp.float32)
    ones16 = jnp.ones((128, 16), jnp.float32)
    mask = jax.random.bernoulli(jax.random.key(42), 0.5,
                                (G, 16)).astype(jnp.float32)
    b1r = b1.reshape(1, 128)
    b2r = b2.reshape(1, 64)
    b3r = b3.reshape(1, 16)
    bcr = bc.reshape(1, 2)

    degp = _sc_degree(ei3, zeros16, ones16)
    h1 = _tc_matmul(xp, W1)
    hs1, dinv = _tc_scale(h1, degp)
    acc1 = _sc_prop(hs1, ei3, 128)
    hs2 = _tc_mid(acc1, hs1, dinv, b1r, W2, 64, jnp.bfloat16)
    acc2 = _sc_prop(hs2, ei3, 64)
    hs3 = _tc_mid(acc2, hs2, dinv, b2r, W3, 16, jnp.float32)
    acc3 = _sc_prop(hs3, ei3, 16)
    out2d, h = _tc_final(acc3, hs3, dinv, b3r, batch2d, mask, Wc, bcr)
    return (out2d.reshape(-1), h)


# speedup vs baseline: 1.3132x; 1.0205x over previous
import functools

import jax
import jax.numpy as jnp
from jax import lax
from jax.experimental import pallas as pl
from jax.experimental.pallas import tpu as pltpu
from jax.experimental.pallas import tpu_sc as plsc

N = 10000
NPAD = 10240
E = 160000
EROWS = E // 128
G = 64
NW = 32
ROWS_PER_W = 40
RPT = NPAD // 16


def _worker_rows(wid):
    nrows = jnp.where(wid < 2, 40, 39)
    base = jnp.where(wid < 2, wid * 40, 80 + (wid - 2) * 39)
    return base, nrows

_mesh = plsc.VectorSubcoreMesh(core_axis_name="c", subcore_axis_name="s")


def _load_worker_rows(ei3_hbm, eiv, base):
    @pl.when(base < 80)
    def _():
        pltpu.sync_copy(ei3_hbm.at[pl.ds(base, 40)], eiv)

    @pl.when(base >= 80)
    def _():
        pltpu.sync_copy(ei3_hbm.at[pl.ds(base, 39)], eiv.at[pl.ds(0, 39)])


def _deg_body(ei3_hbm, zeros_hbm, ones_hbm, out_hbm, eiv, ones_v, acc, sem):
    cid = lax.axis_index("c")
    sid = lax.axis_index("s")
    wid = sid * 2 + cid
    base, nrows = _worker_rows(wid)
    pltpu.sync_copy(zeros_hbm.at[pl.ds(sid * RPT, RPT)],
                    acc.at[pl.ds(sid * RPT, RPT)])
    pltpu.sync_copy(ones_hbm, ones_v)
    _load_worker_rows(ei3_hbm, eiv, base)
    plsc.subcore_barrier()

    def body(j, carry):
        pltpu.sync_copy(ones_v, acc.at[eiv.at[j, 1]], add=True)
        return carry

    lax.fori_loop(0, nrows, body, 0)
    plsc.subcore_barrier()
    pltpu.sync_copy(acc.at[pl.ds(sid * RPT, RPT)],
                    out_hbm.at[cid, pl.ds(sid * RPT, RPT)])


def _sc_degree(ei3, zeros, ones):
    return pl.kernel(
        _deg_body,
        out_type=jax.ShapeDtypeStruct((2, NPAD, 16), jnp.float32),
        mesh=_mesh,
        scratch_types=[
            pltpu.VMEM((ROWS_PER_W, 2, 128), jnp.int32),
            pltpu.VMEM((128, 16), jnp.float32),
            pltpu.VMEM_SHARED((NPAD, 16), jnp.float32),
            pltpu.SemaphoreType.DMA,
        ],
        compiler_params=pltpu.CompilerParams(use_tc_tiling_on_sc=False),
    )(ei3, zeros, ones)


_NG = 8


def _prop_body(hs_hbm, ei3_hbm, out_hbm, eiv, rbufs, acc, sg):
    cid = lax.axis_index("c")
    sid = lax.axis_index("s")
    wid = sid * 2 + cid
    base, nrows = _worker_rows(wid)
    _load_worker_rows(ei3_hbm, eiv, base)
    for b in range(_NG):
        pltpu.async_copy(hs_hbm.at[eiv.at[b, 0]], rbufs[b], sg[b])
    pltpu.sync_copy(hs_hbm.at[pl.ds(sid * RPT, RPT)],
                    acc.at[pl.ds(sid * RPT, RPT)])
    plsc.subcore_barrier()

    def one_op(j, b):
        pltpu.make_async_copy(hs_hbm.at[eiv.at[j, 0]], rbufs[b],
                              sg[b]).wait()
        pltpu.sync_copy(rbufs[b], acc.at[eiv.at[j, 1]], add=True)

        @pl.when(j + _NG < nrows)
        def _():
            pltpu.async_copy(hs_hbm.at[eiv.at[j + _NG, 0]], rbufs[b], sg[b])

    def body(i, carry):
        for b in range(_NG):
            one_op(_NG * i + b, b)
        return carry

    lax.fori_loop(0, ROWS_PER_W // _NG - 1, body, 0)
    for b in range(_NG - 1):
        one_op(ROWS_PER_W - _NG + b, b)

    @pl.when(nrows >= ROWS_PER_W)
    def _():
        one_op(ROWS_PER_W - 1, _NG - 1)

    plsc.subcore_barrier()
    pltpu.sync_copy(acc.at[pl.ds(sid * RPT, RPT)],
                    out_hbm.at[cid, pl.ds(sid * RPT, RPT)])


def _sc_prop(hs, ei3, D):
    dt = hs.dtype
    return pl.kernel(
        _prop_body,
        out_type=jax.ShapeDtypeStruct((2, NPAD, D), dt),
        mesh=_mesh,
        scratch_types=[
            pltpu.VMEM((ROWS_PER_W, 2, 128), jnp.int32),
            [pltpu.VMEM((128, D), dt) for _ in range(_NG)],
            pltpu.VMEM_SHARED((NPAD, D), dt),
            [pltpu.SemaphoreType.DMA for _ in range(_NG)],
        ],
        compiler_params=pltpu.CompilerParams(use_tc_tiling_on_sc=False),
    )(hs, ei3)


def _mm_body(x_ref, w_ref, out_ref):
    out_ref[...] = jnp.dot(x_ref[...], w_ref[...],
                           preferred_element_type=jnp.float32)


def _tc_matmul(xp, W1):
    return pl.pallas_call(
        _mm_body,
        out_shape=jax.ShapeDtypeStruct((NPAD, 128), jnp.float32),
    )(xp, W1)


def _scale_body(h_ref, deg_ref, hs_ref, dinv_ref):
    deg = deg_ref[0, :, 0:1] + deg_ref[1, :, 0:1] + 1.0
    dinv = lax.rsqrt(deg)
    hs_ref[...] = (h_ref[...] * dinv).astype(jnp.bfloat16)
    dinv_ref[...] = dinv


def _tc_scale(h1, degp):
    return pl.pallas_call(
        _scale_body,
        out_shape=[
            jax.ShapeDtypeStruct((NPAD, 128), jnp.bfloat16),
            jax.ShapeDtypeStruct((NPAD, 1), jnp.float32),
        ],
    )(h1, degp)


def _tc_mid_body(a_ref, hs_ref, dinv_ref, b_ref, w_ref, out_ref):
    dinv = dinv_ref[...]
    a = a_ref[0].astype(jnp.float32) + a_ref[1].astype(jnp.float32)
    p = jnp.tanh(dinv * (a - hs_ref[...].astype(jnp.float32)) + b_ref[...])
    out = jnp.dot(p, w_ref[...], preferred_element_type=jnp.float32) * dinv
    out_ref[...] = out.astype(out_ref.dtype)


def _tc_mid(acc, hs, dinv, b, W, Dout, out_dtype):
    return pl.pallas_call(
        _tc_mid_body,
        out_shape=jax.ShapeDtypeStruct((NPAD, Dout), out_dtype),
    )(acc, hs, dinv, b, W)


def _tc_final_body(a_ref, hs_ref, dinv_ref, b_ref, batch_ref, mask_ref,
                   wc_ref, bc_ref, out_ref, h_ref):
    dinv = dinv_ref[...]
    p = jnp.tanh(dinv * (a_ref[0] + a_ref[1] - hs_ref[...]) + b_ref[...])
    paug = jnp.concatenate([p, jnp.ones((NPAD, 1), jnp.float32)], axis=1)
    iota = lax.broadcasted_iota(jnp.int32, (NPAD, G), 1)
    onehot = (batch_ref[...] == iota).astype(jnp.float32)
    s = lax.dot_general(onehot, paug, (((0,), (0,)), ((), ())),
                        preferred_element_type=jnp.float32)
    hp = s[:, 0:16] / jnp.maximum(s[:, 16:17], 1.0)
    hd = mask_ref[...] * (2.0 * hp)
    h_ref[...] = hd
    out_ref[...] = jnp.dot(hd, wc_ref[...],
                           preferred_element_type=jnp.float32) + bc_ref[...]


def _tc_final(acc, hs, dinv, b, batch2d, mask, Wc, bc):
    return pl.pallas_call(
        _tc_final_body,
        out_shape=[
            jax.ShapeDtypeStruct((G, 2), jnp.float32),
            jax.ShapeDtypeStruct((G, 16), jnp.float32),
        ],
    )(acc, hs, dinv, b, batch2d, mask, Wc, bc)


def kernel(x, edge_index, batch, W1, b1, W2, b2, W3, b3, Wc, bc):
    xp = jnp.zeros((NPAD, 256), jnp.float32).at[:N].set(x)
    ei3 = jnp.transpose(
        edge_index.astype(jnp.int32).reshape(2, EROWS, 128), (1, 0, 2))
    batch2d = jnp.concatenate(
        [batch.astype(jnp.int32),
         jnp.full((NPAD - N,), G, jnp.int32)]).reshape(NPAD, 1)
    zeros16 = jnp.zeros((NPAD, 16), jnp.float32)
    ones16 = jnp.ones((128, 16), jnp.float32)
    mask = jax.random.bernoulli(jax.random.key(42), 0.5,
                                (G, 16)).astype(jnp.float32)
    b1r = b1.reshape(1, 128)
    b2r = b2.reshape(1, 64)
    b3r = b3.reshape(1, 16)
    bcr = bc.reshape(1, 2)

    degp = _sc_degree(ei3, zeros16, ones16)
    h1 = _tc_matmul(xp, W1)
    hs1, dinv = _tc_scale(h1, degp)
    acc1 = _sc_prop(hs1, ei3, 128)
    hs2 = _tc_mid(acc1, hs1, dinv, b1r, W2, 64, jnp.bfloat16)
    acc2 = _sc_prop(hs2, ei3, 64)
    hs3 = _tc_mid(acc2, hs2, dinv, b2r, W3, 16, jnp.float32)
    acc3 = _sc_prop(hs3, ei3, 16)
    out2d, h = _tc_final(acc3, hs3, dinv, b3r, batch2d, mask, Wc, bcr)
    return (out2d.reshape(-1), h)
